# baseline (speedup 1.0000x reference)
import functools

import jax
import jax.numpy as jnp
from jax import lax
from jax.experimental import pallas as pl
from jax.experimental.pallas import tpu as pltpu
from jax.experimental.pallas import tpu_sc as plsc

N = 10000
E = 160000
D = 256
R = 8
B = 8
H = 128

NSUB = 16
CHUNK = 128
CH = 79
EPAD = NSUB * CH * CHUNK
TRASH = N
ACC_ROWS = N + 8
MB = 10000
NMB = N // MB
ZB = 1000


IB = EPAD // CHUNK


def _front1_body(x_ref, win_ref, bin_ref, c1_ref, bs1_ref, c2_ref, bs2_ref,
                 src_ref, et_ref,
                 lo_ref, hi_ref, h_ref, gidx_ref, w2_ref, w_s):
    m = pl.program_id(0)
    r = pl.program_id(1)

    @pl.when(jnp.logical_and(m == 0, r == 0))
    def _():
        w1v = jnp.dot(c1_ref[...].astype(jnp.bfloat16),
                      bs1_ref[...].astype(jnp.bfloat16),
                      preferred_element_type=jnp.float32)
        for rr in range(R):
            w_s[rr] = w1v[rr].astype(jnp.bfloat16).reshape(D, D)
        w2_ref[...] = jnp.dot(c2_ref[...].astype(jnp.bfloat16),
                              bs2_ref[...].astype(jnp.bfloat16),
                              preferred_element_type=jnp.float32)
        gidx_ref[...] = et_ref[...] * N + src_ref[...]

    @pl.when(r == 0)
    def _():
        xb = x_ref[...].astype(jnp.bfloat16)
        wb = win_ref[...].astype(jnp.bfloat16)
        h_ref[...] = (jnp.dot(xb, wb, preferred_element_type=jnp.float32)
                      + bin_ref[...])

    hb = h_ref[...].astype(jnp.bfloat16)
    o = jnp.dot(hb, w_s[r], preferred_element_type=jnp.float32)
    lo_ref[...] = o[:, :H]
    hi_ref[...] = o[:, H:]


def _front1(x, W_in, b_in2, coef1, bases1f, coef2, bases2f, src2, et2):
    return pl.pallas_call(
        _front1_body,
        grid=(NMB, R),
        in_specs=[
            pl.BlockSpec((MB, D), lambda m, r: (m, 0)),
            pl.BlockSpec((D, D), lambda m, r: (0, 0)),
            pl.BlockSpec((1, D), lambda m, r: (0, 0)),
            pl.BlockSpec((R, B), lambda m, r: (0, 0)),
            pl.BlockSpec((B, D * D), lambda m, r: (0, 0)),
            pl.BlockSpec((R, B), lambda m, r: (0, 0)),
            pl.BlockSpec((B, D * D), lambda m, r: (0, 0)),
            pl.BlockSpec((IB, CHUNK), lambda m, r: (0, 0)),
            pl.BlockSpec((IB, CHUNK), lambda m, r: (0, 0)),
        ],
        out_specs=[
            pl.BlockSpec((MB, H), lambda m, r: (r * NMB + m, 0)),
            pl.BlockSpec((MB, H), lambda m, r: (r * NMB + m, 0)),
            pl.BlockSpec((MB, D), lambda m, r: (m, 0)),
            pl.BlockSpec((IB, CHUNK), lambda m, r: (0, 0)),
            pl.BlockSpec((R, D * D), lambda m, r: (0, 0)),
        ],
        out_shape=[
            jax.ShapeDtypeStruct((R * N, H), jnp.float32),
            jax.ShapeDtypeStruct((R * N, H), jnp.float32),
            jax.ShapeDtypeStruct((N, D), jnp.float32),
            jax.ShapeDtypeStruct((IB, CHUNK), jnp.int32),
            jax.ShapeDtypeStruct((R, D * D), jnp.float32),
        ],
        scratch_shapes=[pltpu.VMEM((R, D, D), jnp.bfloat16)],
    )(x, W_in, b_in2, coef1, bases1f, coef2, bases2f, src2, et2)


def _front2_body(h_ref, w_ref, lo_ref, hi_ref, w_s):
    m = pl.program_id(0)
    r = pl.program_id(1)

    @pl.when(m == 0)
    def _():
        w_s[r] = w_ref[r].astype(jnp.bfloat16).reshape(D, D)

    hb = h_ref[...].astype(jnp.bfloat16)
    o = jnp.dot(hb, w_s[r], preferred_element_type=jnp.float32)
    lo_ref[...] = o[:, :H]
    hi_ref[...] = o[:, H:]


def _front2(h, w2d):
    return pl.pallas_call(
        _front2_body,
        grid=(NMB, R),
        in_specs=[
            pl.BlockSpec((MB, D), lambda m, r: (m, 0)),
            pl.BlockSpec((R, D * D), lambda m, r: (0, 0)),
        ],
        out_specs=[
            pl.BlockSpec((MB, H), lambda m, r: (r * NMB + m, 0)),
            pl.BlockSpec((MB, H), lambda m, r: (r * NMB + m, 0)),
        ],
        out_shape=[
            jax.ShapeDtypeStruct((R * N, H), jnp.float32),
            jax.ShapeDtypeStruct((R * N, H), jnp.float32),
        ],
        scratch_shapes=[pltpu.VMEM((R, D, D), jnp.bfloat16)],
    )(h, w2d)


def _ln_body(lo_ref, hi_ref, h_ref, ws_ref, bias_ref, g_ref, b_ref, o_ref, *,
             relu):
    hb = h_ref[...].astype(jnp.bfloat16)
    wb = ws_ref[...].astype(jnp.bfloat16)
    s = jnp.dot(hb, wb, preferred_element_type=jnp.float32)
    o = jnp.concatenate([lo_ref[...], hi_ref[...]], axis=1) + s + bias_ref[...]
    mu = jnp.mean(o, axis=1, keepdims=True)
    d = o - mu
    var = jnp.mean(d * d, axis=1, keepdims=True)
    y = d * lax.rsqrt(var + 1e-5) * g_ref[...] + b_ref[...]
    if relu:
        y = jnp.maximum(y, 0.0)
    o_ref[...] = y


def _ln(agg_lo, agg_hi, h, Wself, bias2, g2, b2, relu):
    return pl.pallas_call(
        functools.partial(_ln_body, relu=relu),
        grid=(NMB,),
        in_specs=[
            pl.BlockSpec((MB, H), lambda m: (m, 0)),
            pl.BlockSpec((MB, H), lambda m: (m, 0)),
            pl.BlockSpec((MB, D), lambda m: (m, 0)),
            pl.BlockSpec((D, D), lambda m: (0, 0)),
            pl.BlockSpec((1, D), lambda m: (0, 0)),
            pl.BlockSpec((1, D), lambda m: (0, 0)),
            pl.BlockSpec((1, D), lambda m: (0, 0)),
        ],
        out_specs=pl.BlockSpec((MB, D), lambda m: (m, 0)),
        out_shape=jax.ShapeDtypeStruct((N, D), jnp.float32),
    )(agg_lo, agg_hi, h, Wself, bias2, g2, b2)


def _sc_agg_body(hlo_hbm, hhi_hbm, gidx_hbm, dst_hbm, zer_hbm,
                 outlo_hbm, outhi_hbm,
                 gidx_v, dst_v, rows_v, acc, sem):
    c = lax.axis_index("c")
    s = lax.axis_index("s")

    @pl.when(s < 10)
    def _():
        pltpu.sync_copy(zer_hbm, acc.at[pl.ds(s * ZB, ZB)])

    @pl.when(s == 10)
    def _():
        pltpu.sync_copy(zer_hbm.at[pl.ds(0, 8)], acc.at[pl.ds(N, 8)])

    pltpu.sync_copy(gidx_hbm.at[s], gidx_v)
    pltpu.sync_copy(dst_hbm.at[s], dst_v)
    plsc.subcore_barrier()

    def edge_loop(table):
        @pl.loop(0, CH)
        def _(j):
            pltpu.async_copy(table.at[gidx_v.at[j]], rows_v, sem).wait()
            pltpu.sync_copy(rows_v, acc.at[dst_v.at[j]], add=True)

    @pl.when(c == 0)
    def _():
        edge_loop(hlo_hbm)

    @pl.when(c == 1)
    def _():
        edge_loop(hhi_hbm)

    plsc.subcore_barrier()

    @pl.when(jnp.logical_and(s < 10, c == 0))
    def _():
        pltpu.sync_copy(acc.at[pl.ds(s * ZB, ZB)], outlo_hbm.at[pl.ds(s * ZB, ZB)])

    @pl.when(jnp.logical_and(s < 10, c == 1))
    def _():
        pltpu.sync_copy(acc.at[pl.ds(s * ZB, ZB)], outhi_hbm.at[pl.ds(s * ZB, ZB)])


@functools.cache
def _sc_agg_kernel():
    mesh = plsc.VectorSubcoreMesh(core_axis_name="c", subcore_axis_name="s",
                                  num_cores=2, num_subcores=NSUB)
    return pl.kernel(
        _sc_agg_body,
        out_type=(
            jax.ShapeDtypeStruct((N, H), jnp.float32),
            jax.ShapeDtypeStruct((N, H), jnp.float32),
        ),
        mesh=mesh,
        scratch_types=[
            pltpu.VMEM((CH, CHUNK), jnp.int32),
            pltpu.VMEM((CH, CHUNK), jnp.int32),
            pltpu.VMEM((CHUNK, H), jnp.float32),
            pltpu.VMEM_SHARED((ACC_ROWS, H), jnp.float32),
            pltpu.SemaphoreType.DMA,
        ],
    )


def _sc_agg(hlo, hhi, gidx3, dst3, zer):
    return _sc_agg_kernel()(hlo, hhi, gidx3, dst3, zer)


def kernel(x, edge_index, etypes, W_in, b_in, bases1, coef1, Wself1, bias1,
           ln1_g, ln1_b, bases2, coef2, Wself2, bias2, ln2_g, ln2_b):
    pad = EPAD - E
    src_p = jnp.pad(edge_index[0].astype(jnp.int32), (0, pad))
    et_p = jnp.pad(etypes.astype(jnp.int32), (0, pad))
    dst_p = jnp.pad(edge_index[1].astype(jnp.int32), (0, pad),
                    constant_values=TRASH)
    src2 = src_p.reshape(IB, CHUNK)
    et2 = et_p.reshape(IB, CHUNK)
    dst3 = dst_p.reshape(NSUB, CH, CHUNK)
    zer = jnp.zeros((ZB, H), jnp.float32)

    hlo, hhi, h, gidx2, w2 = _front1(
        x, W_in, b_in.reshape(1, D), coef1, bases1.reshape(B, D * D),
        coef2, bases2.reshape(B, D * D), src2, et2)
    gidx3 = gidx2.reshape(NSUB, CH, CHUNK)
    agg_lo, agg_hi = _sc_agg(hlo, hhi, gidx3, dst3, zer)
    h = _ln(agg_lo, agg_hi, h, Wself1, bias1.reshape(1, D),
            ln1_g.reshape(1, D), ln1_b.reshape(1, D), True)

    hlo, hhi = _front2(h, w2)
    agg_lo, agg_hi = _sc_agg(hlo, hhi, gidx3, dst3, zer)
    h = _ln(agg_lo, agg_hi, h, Wself2, bias2.reshape(1, D),
            ln2_g.reshape(1, D), ln2_b.reshape(1, D), False)
    return h


# speedup vs baseline: 1.0043x; 1.0043x over previous
import functools

import jax
import jax.numpy as jnp
from jax import lax
from jax.experimental import pallas as pl
from jax.experimental.pallas import tpu as pltpu
from jax.experimental.pallas import tpu_sc as plsc

N = 10000
E = 160000
D = 256
R = 8
B = 8
H = 128

NSUB = 16
CHUNK = 128
CH = 79
EPAD = NSUB * CH * CHUNK
TRASH = N
ACC_ROWS = N + 8
MB = 5000
NMB = N // MB
ZB = 1000


IB = EPAD // CHUNK


def _front1_body(x_ref, win_ref, bin_ref, c1_ref, bs1_ref, c2_ref, bs2_ref,
                 src_ref, et_ref,
                 lo_ref, hi_ref, h_ref, gidx_ref, w2_ref, w_s):
    m = pl.program_id(0)
    r = pl.program_id(1)

    @pl.when(jnp.logical_and(m == 0, r == 0))
    def _():
        w1v = jnp.dot(c1_ref[...].astype(jnp.bfloat16),
                      bs1_ref[...].astype(jnp.bfloat16),
                      preferred_element_type=jnp.float32)
        for rr in range(R):
            w_s[rr] = w1v[rr].astype(jnp.bfloat16).reshape(D, D)
        w2_ref[...] = jnp.dot(c2_ref[...].astype(jnp.bfloat16),
                              bs2_ref[...].astype(jnp.bfloat16),
                              preferred_element_type=jnp.float32)
        gidx_ref[...] = et_ref[...] * N + src_ref[...]

    @pl.when(r == 0)
    def _():
        xb = x_ref[...].astype(jnp.bfloat16)
        wb = win_ref[...].astype(jnp.bfloat16)
        h_ref[...] = (jnp.dot(xb, wb, preferred_element_type=jnp.float32)
                      + bin_ref[...])

    hb = h_ref[...].astype(jnp.bfloat16)
    o = jnp.dot(hb, w_s[r], preferred_element_type=jnp.float32)
    lo_ref[...] = o[:, :H]
    hi_ref[...] = o[:, H:]


def _front1(x, W_in, b_in2, coef1, bases1f, coef2, bases2f, src2, et2):
    return pl.pallas_call(
        _front1_body,
        grid=(NMB, R),
        in_specs=[
            pl.BlockSpec((MB, D), lambda m, r: (m, 0)),
            pl.BlockSpec((D, D), lambda m, r: (0, 0)),
            pl.BlockSpec((1, D), lambda m, r: (0, 0)),
            pl.BlockSpec((R, B), lambda m, r: (0, 0)),
            pl.BlockSpec((B, D * D), lambda m, r: (0, 0)),
            pl.BlockSpec((R, B), lambda m, r: (0, 0)),
            pl.BlockSpec((B, D * D), lambda m, r: (0, 0)),
            pl.BlockSpec((IB, CHUNK), lambda m, r: (0, 0)),
            pl.BlockSpec((IB, CHUNK), lambda m, r: (0, 0)),
        ],
        out_specs=[
            pl.BlockSpec((MB, H), lambda m, r: (r * NMB + m, 0)),
            pl.BlockSpec((MB, H), lambda m, r: (r * NMB + m, 0)),
            pl.BlockSpec((MB, D), lambda m, r: (m, 0)),
            pl.BlockSpec((IB, CHUNK), lambda m, r: (0, 0)),
            pl.BlockSpec((R, D * D), lambda m, r: (0, 0)),
        ],
        out_shape=[
            jax.ShapeDtypeStruct((R * N, H), jnp.float32),
            jax.ShapeDtypeStruct((R * N, H), jnp.float32),
            jax.ShapeDtypeStruct((N, D), jnp.float32),
            jax.ShapeDtypeStruct((IB, CHUNK), jnp.int32),
            jax.ShapeDtypeStruct((R, D * D), jnp.float32),
        ],
        scratch_shapes=[pltpu.VMEM((R, D, D), jnp.bfloat16)],
    )(x, W_in, b_in2, coef1, bases1f, coef2, bases2f, src2, et2)


def _front2_body(h_ref, w_ref, lo_ref, hi_ref, w_s):
    m = pl.program_id(0)
    r = pl.program_id(1)

    @pl.when(m == 0)
    def _():
        w_s[r] = w_ref[r].astype(jnp.bfloat16).reshape(D, D)

    hb = h_ref[...].astype(jnp.bfloat16)
    o = jnp.dot(hb, w_s[r], preferred_element_type=jnp.float32)
    lo_ref[...] = o[:, :H]
    hi_ref[...] = o[:, H:]


def _front2(h, w2d):
    return pl.pallas_call(
        _front2_body,
        grid=(NMB, R),
        in_specs=[
            pl.BlockSpec((MB, D), lambda m, r: (m, 0)),
            pl.BlockSpec((R, D * D), lambda m, r: (0, 0)),
        ],
        out_specs=[
            pl.BlockSpec((MB, H), lambda m, r: (r * NMB + m, 0)),
            pl.BlockSpec((MB, H), lambda m, r: (r * NMB + m, 0)),
        ],
        out_shape=[
            jax.ShapeDtypeStruct((R * N, H), jnp.float32),
            jax.ShapeDtypeStruct((R * N, H), jnp.float32),
        ],
        scratch_shapes=[pltpu.VMEM((R, D, D), jnp.bfloat16)],
    )(h, w2d)


def _ln_body(lo_ref, hi_ref, h_ref, ws_ref, bias_ref, g_ref, b_ref, o_ref, *,
             relu):
    hb = h_ref[...].astype(jnp.bfloat16)
    wb = ws_ref[...].astype(jnp.bfloat16)
    s = jnp.dot(hb, wb, preferred_element_type=jnp.float32)
    o = jnp.concatenate([lo_ref[...], hi_ref[...]], axis=1) + s + bias_ref[...]
    mu = jnp.mean(o, axis=1, keepdims=True)
    d = o - mu
    var = jnp.mean(d * d, axis=1, keepdims=True)
    y = d * lax.rsqrt(var + 1e-5) * g_ref[...] + b_ref[...]
    if relu:
        y = jnp.maximum(y, 0.0)
    o_ref[...] = y


def _ln(agg_lo, agg_hi, h, Wself, bias2, g2, b2, relu):
    return pl.pallas_call(
        functools.partial(_ln_body, relu=relu),
        grid=(NMB,),
        in_specs=[
            pl.BlockSpec((MB, H), lambda m: (m, 0)),
            pl.BlockSpec((MB, H), lambda m: (m, 0)),
            pl.BlockSpec((MB, D), lambda m: (m, 0)),
            pl.BlockSpec((D, D), lambda m: (0, 0)),
            pl.BlockSpec((1, D), lambda m: (0, 0)),
            pl.BlockSpec((1, D), lambda m: (0, 0)),
            pl.BlockSpec((1, D), lambda m: (0, 0)),
        ],
        out_specs=pl.BlockSpec((MB, D), lambda m: (m, 0)),
        out_shape=jax.ShapeDtypeStruct((N, D), jnp.float32),
    )(agg_lo, agg_hi, h, Wself, bias2, g2, b2)


def _sc_agg_body(hlo_hbm, hhi_hbm, gidx_hbm, dst_hbm, zer_hbm,
                 outlo_hbm, outhi_hbm,
                 gidx_v, dst_v, rows_v, acc, sem):
    c = lax.axis_index("c")
    s = lax.axis_index("s")

    @pl.when(s < 10)
    def _():
        pltpu.sync_copy(zer_hbm, acc.at[pl.ds(s * ZB, ZB)])

    @pl.when(s == 10)
    def _():
        pltpu.sync_copy(zer_hbm.at[pl.ds(0, 8)], acc.at[pl.ds(N, 8)])

    pltpu.sync_copy(gidx_hbm.at[s], gidx_v)
    pltpu.sync_copy(dst_hbm.at[s], dst_v)
    plsc.subcore_barrier()

    def edge_loop(table):
        @pl.loop(0, CH)
        def _(j):
            pltpu.async_copy(table.at[gidx_v.at[j]], rows_v, sem).wait()
            pltpu.sync_copy(rows_v, acc.at[dst_v.at[j]], add=True)

    @pl.when(c == 0)
    def _():
        edge_loop(hlo_hbm)

    @pl.when(c == 1)
    def _():
        edge_loop(hhi_hbm)

    plsc.subcore_barrier()

    @pl.when(jnp.logical_and(s < 10, c == 0))
    def _():
        pltpu.sync_copy(acc.at[pl.ds(s * ZB, ZB)], outlo_hbm.at[pl.ds(s * ZB, ZB)])

    @pl.when(jnp.logical_and(s < 10, c == 1))
    def _():
        pltpu.sync_copy(acc.at[pl.ds(s * ZB, ZB)], outhi_hbm.at[pl.ds(s * ZB, ZB)])


@functools.cache
def _sc_agg_kernel():
    mesh = plsc.VectorSubcoreMesh(core_axis_name="c", subcore_axis_name="s",
                                  num_cores=2, num_subcores=NSUB)
    return pl.kernel(
        _sc_agg_body,
        out_type=(
            jax.ShapeDtypeStruct((N, H), jnp.float32),
            jax.ShapeDtypeStruct((N, H), jnp.float32),
        ),
        mesh=mesh,
        scratch_types=[
            pltpu.VMEM((CH, CHUNK), jnp.int32),
            pltpu.VMEM((CH, CHUNK), jnp.int32),
            pltpu.VMEM((CHUNK, H), jnp.float32),
            pltpu.VMEM_SHARED((ACC_ROWS, H), jnp.float32),
            pltpu.SemaphoreType.DMA,
        ],
    )


def _sc_agg(hlo, hhi, gidx3, dst3, zer):
    return _sc_agg_kernel()(hlo, hhi, gidx3, dst3, zer)


def kernel(x, edge_index, etypes, W_in, b_in, bases1, coef1, Wself1, bias1,
           ln1_g, ln1_b, bases2, coef2, Wself2, bias2, ln2_g, ln2_b):
    pad = EPAD - E
    src_p = jnp.pad(edge_index[0].astype(jnp.int32), (0, pad))
    et_p = jnp.pad(etypes.astype(jnp.int32), (0, pad))
    dst_p = jnp.pad(edge_index[1].astype(jnp.int32), (0, pad),
                    constant_values=TRASH)
    src2 = src_p.reshape(IB, CHUNK)
    et2 = et_p.reshape(IB, CHUNK)
    dst3 = dst_p.reshape(NSUB, CH, CHUNK)
    zer = jnp.zeros((ZB, H), jnp.float32)

    hlo, hhi, h, gidx2, w2 = _front1(
        x, W_in, b_in.reshape(1, D), coef1, bases1.reshape(B, D * D),
        coef2, bases2.reshape(B, D * D), src2, et2)
    gidx3 = gidx2.reshape(NSUB, CH, CHUNK)
    agg_lo, agg_hi = _sc_agg(hlo, hhi, gidx3, dst3, zer)
    h = _ln(agg_lo, agg_hi, h, Wself1, bias1.reshape(1, D),
            ln1_g.reshape(1, D), ln1_b.reshape(1, D), True)

    hlo, hhi = _front2(h, w2)
    agg_lo, agg_hi = _sc_agg(hlo, hhi, gidx3, dst3, zer)
    h = _ln(agg_lo, agg_hi, h, Wself2, bias2.reshape(1, D),
            ln2_g.reshape(1, D), ln2_b.reshape(1, D), False)
    return h
